# baseline (speedup 1.0000x reference)
import functools

import jax
import jax.numpy as jnp
from jax import lax
from jax.experimental import pallas as pl
from jax.experimental.pallas import tpu as pltpu
from jax.experimental.pallas import tpu_sc as plsc

NC = 2
NS = 16
NW = NC * NS
LANES = 16
CHUNK = 128


def _make_spmm(n, e, f):
    epw = e // NW
    C = CHUNK
    nchunk = epw // C
    n_pad = -(-n // 128) * 128
    rpw = n_pad // NS
    mesh = plsc.VectorSubcoreMesh(core_axis_name="c", subcore_axis_name="s")

    @functools.partial(
        pl.kernel,
        out_type=jax.ShapeDtypeStruct((2 * n_pad, f), jnp.float32),
        mesh=mesh,
        compiler_params=pltpu.CompilerParams(needs_layout_passes=False),
        scratch_types=[
            pltpu.VMEM((C,), jnp.int32),
            pltpu.VMEM((C,), jnp.int32),
            pltpu.VMEM((C,), jnp.float32),
            pltpu.VMEM((C,), jnp.int32),
            pltpu.VMEM((C,), jnp.int32),
            pltpu.VMEM((C,), jnp.float32),
            pltpu.VMEM((C, f), jnp.float32),
            pltpu.VMEM((C, f), jnp.float32),
            pltpu.VMEM_SHARED((n_pad, f), jnp.float32),
            pltpu.SemaphoreType.DMA,
            pltpu.SemaphoreType.DMA,
            pltpu.SemaphoreType.DMA,
            pltpu.SemaphoreType.DMA,
            pltpu.SemaphoreType.DMA,
        ],
    )
    def spmm(table, cols_h, rows_h, vals_h, out, cv0, rv0, vv0,
             cv1, rv1, vv1, g0, g1, acc, sem, is0, is1, ss0, ss1):
        c = lax.axis_index("c")
        s = lax.axis_index("s")
        wid = s * NC + c

        z16 = jnp.zeros((LANES,), jnp.float32)

        def zrow(r, carry):
            for j in range(f // LANES):
                g0[r, pl.ds(j * LANES, LANES)] = z16
            return carry

        lax.fori_loop(0, C, zrow, 0)
        nzc = rpw // C
        rem = rpw - nzc * C
        for q in range(nzc):
            pltpu.sync_copy(g0, acc.at[pl.ds(s * rpw + q * C, C)])
        if rem:
            pltpu.sync_copy(g0.at[pl.ds(0, rem)],
                            acc.at[pl.ds(s * rpw + nzc * C, rem)])
        plsc.subcore_barrier()

        base = wid * epw
        colb = (cv0, cv1)
        rowb = (rv0, rv1)
        valb = (vv0, vv1)
        isems = (is0, is1)

        def fetch_idx(i, b):
            off = base + i * C
            pltpu.async_copy(cols_h.at[pl.ds(off, C)], colb[b], isems[b])
            pltpu.async_copy(rows_h.at[pl.ds(off, C)], rowb[b], isems[b])
            pltpu.async_copy(vals_h.at[pl.ds(off, C)], valb[b], isems[b])

        def wait_idx(b):
            for dst in (colb[b], rowb[b], valb[b]):
                pltpu.make_async_copy(cols_h.at[pl.ds(0, C)], dst,
                                      isems[b]).wait()

        gbufs = (g0, g1)
        ssems = (ss0, ss1)

        def scale_chunk(valv, g):
            nsl = f // LANES

            def grp_body(grp, carry):
                e0 = grp * 2
                e1 = e0 + 1
                vw0 = plsc.load_gather(valv,
                                       [jnp.broadcast_to(e0, (LANES,))])
                vw1 = plsc.load_gather(valv,
                                       [jnp.broadcast_to(e1, (LANES,))])
                t0 = [g[e0, pl.ds(j * LANES, LANES)] for j in range(nsl)]
                t1 = [g[e1, pl.ds(j * LANES, LANES)] for j in range(nsl)]
                for j in range(nsl):
                    g[e0, pl.ds(j * LANES, LANES)] = t0[j] * vw0
                for j in range(nsl):
                    g[e1, pl.ds(j * LANES, LANES)] = t1[j] * vw1
                return carry

            lax.fori_loop(0, C // 2, grp_body, 0)

        fetch_idx(0, 0)

        def half_body(p, b, i):
            @pl.when(p >= 1)
            def _():
                pltpu.make_async_copy(gbufs[b], acc.at[rowb[b]],
                                      ssems[b]).wait()
            wait_idx(b)
            pltpu.async_copy(table.at[colb[b]], gbufs[b], sem).wait()
            if b == 0:
                fetch_idx(i + 1, 1)
            else:
                @pl.when(i + 1 < nchunk)
                def _():
                    fetch_idx(i + 1, 0)
            scale_chunk(valb[b], gbufs[b])
            pltpu.async_copy(gbufs[b], acc.at[rowb[b]], ssems[b], add=True)

        def pair_body(p, carry):
            half_body(p, 0, 2 * p)
            half_body(p, 1, 2 * p + 1)
            return carry

        lax.fori_loop(0, nchunk // 2, pair_body, 0)
        if nchunk % 2:
            pltpu.make_async_copy(g0, acc.at[rowb[0]], ss0).wait()
            wait_idx(0)
            pltpu.async_copy(table.at[colb[0]], g0, sem).wait()
            scale_chunk(valb[0], g0)
            pltpu.sync_copy(g0, acc.at[rowb[0]], add=True)
            pltpu.make_async_copy(g1, acc.at[rowb[1]], ss1).wait()
        else:
            pltpu.make_async_copy(g0, acc.at[rowb[0]], ss0).wait()
            pltpu.make_async_copy(g1, acc.at[rowb[1]], ss1).wait()
        plsc.subcore_barrier()

        pltpu.sync_copy(acc.at[pl.ds(s * rpw, rpw)],
                        out.at[pl.ds(c * n_pad + s * rpw, rpw)])

    return spmm, n_pad


def _tc1(x, p0, p1, w0, w1, inv_n):
    n, f = x.shape
    blk = 1000

    def body(xr, p0r, p1r, w0r, w1r, z1r, y01r):
        z1 = (p0r[...] + p1r[...]) * inv_n
        z1r[...] = z1
        y01r[...] = (jnp.dot(xr[...], w0r[...],
                             preferred_element_type=jnp.float32)
                     + jnp.dot(z1, w1r[...],
                               preferred_element_type=jnp.float32))

    row_spec = pl.BlockSpec((blk, f), lambda i: (i, 0))
    w_spec = pl.BlockSpec((f, f), lambda i: (0, 0))
    return pl.pallas_call(
        body,
        grid=(n // blk,),
        in_specs=[row_spec, row_spec, row_spec, w_spec, w_spec],
        out_specs=[row_spec, row_spec],
        out_shape=[jax.ShapeDtypeStruct((n, f), jnp.float32),
                   jax.ShapeDtypeStruct((n, f), jnp.float32)],
    )(x, p0, p1, w0, w1)


def _tc2(y01, q0, q1, w2, inv_n):
    n, f = y01.shape
    blk = 1000

    def body(y01r, q0r, q1r, w2r, yr):
        z2 = (q0r[...] + q1r[...]) * inv_n
        yr[...] = y01r[...] + jnp.dot(z2, w2r[...],
                                      preferred_element_type=jnp.float32)

    row_spec = pl.BlockSpec((blk, f), lambda i: (i, 0))
    w_spec = pl.BlockSpec((f, f), lambda i: (0, 0))
    return pl.pallas_call(
        body,
        grid=(n // blk,),
        in_specs=[row_spec, row_spec, row_spec, w_spec],
        out_specs=row_spec,
        out_shape=jax.ShapeDtypeStruct((n, f), jnp.float32),
    )(y01, q0, q1, w2)


def kernel(x, edge_index, edge_weight, weights):
    n, f = x.shape
    e = edge_weight.shape[0]
    rows = edge_index[0]
    cols = edge_index[1]
    inv_n = float(1.0 / n)

    unit = NW * CHUNK * 2
    e_pad = -(-e // unit) * unit
    pad = e_pad - e
    cols = jnp.concatenate([cols, jnp.zeros((pad,), cols.dtype)])
    rows = jnp.concatenate([rows, jnp.zeros((pad,), rows.dtype)])
    edge_weight = jnp.concatenate(
        [edge_weight, jnp.zeros((pad,), edge_weight.dtype)])

    spmm, n_pad = _make_spmm(n, e_pad, f)
    p = spmm(x, cols, rows, edge_weight)
    z1, y01 = _tc1(x, p[:n], p[n_pad:n_pad + n], weights[0], weights[1],
                   inv_n)
    q = spmm(z1, cols, rows, edge_weight)
    return _tc2(y01, q[:n], q[n_pad:n_pad + n], weights[2], inv_n)


# speedup vs baseline: 1.3349x; 1.3349x over previous
import functools

import jax
import jax.numpy as jnp
from jax import lax
from jax.experimental import pallas as pl
from jax.experimental.pallas import tpu as pltpu
from jax.experimental.pallas import tpu_sc as plsc

NC = 2
NS = 16
NW = NC * NS
LANES = 16
CHUNK = 64


def _make_spmm(n, e, f):
    epw = e // NW
    C = CHUNK
    nchunk = epw // C
    n_pad = -(-n // 128) * 128
    rpw = n_pad // NS
    mesh = plsc.VectorSubcoreMesh(core_axis_name="c", subcore_axis_name="s")

    @functools.partial(
        pl.kernel,
        out_type=jax.ShapeDtypeStruct((2 * n_pad, f), jnp.float32),
        mesh=mesh,
        compiler_params=pltpu.CompilerParams(needs_layout_passes=False),
        scratch_types=[
            pltpu.VMEM((C,), jnp.int32),
            pltpu.VMEM((C,), jnp.int32),
            pltpu.VMEM((C,), jnp.float32),
            pltpu.VMEM((C,), jnp.int32),
            pltpu.VMEM((C,), jnp.int32),
            pltpu.VMEM((C,), jnp.float32),
            pltpu.VMEM((C, f), jnp.float32),
            pltpu.VMEM((C, f), jnp.float32),
            pltpu.VMEM_SHARED((n_pad, f), jnp.float32),
            pltpu.SemaphoreType.DMA,
            pltpu.SemaphoreType.DMA,
            pltpu.SemaphoreType.DMA,
            pltpu.SemaphoreType.DMA,
            pltpu.SemaphoreType.DMA,
        ],
    )
    def spmm(table, cols_h, rows_h, vals_h, out, cv0, rv0, vv0,
             cv1, rv1, vv1, g0, g1, acc, sem, is0, is1, ss0, ss1):
        c = lax.axis_index("c")
        s = lax.axis_index("s")
        wid = s * NC + c

        z16 = jnp.zeros((LANES,), jnp.float32)

        def zrow(r, carry):
            for j in range(f // LANES):
                g0[r, pl.ds(j * LANES, LANES)] = z16
            return carry

        lax.fori_loop(0, C, zrow, 0)
        nzc = rpw // C
        rem = rpw - nzc * C
        for q in range(nzc):
            pltpu.sync_copy(g0, acc.at[pl.ds(s * rpw + q * C, C)])
        if rem:
            pltpu.sync_copy(g0.at[pl.ds(0, rem)],
                            acc.at[pl.ds(s * rpw + nzc * C, rem)])
        plsc.subcore_barrier()

        base = wid * epw
        colb = (cv0, cv1)
        rowb = (rv0, rv1)
        valb = (vv0, vv1)
        isems = (is0, is1)

        def fetch_idx(i, b):
            off = base + i * C
            pltpu.async_copy(cols_h.at[pl.ds(off, C)], colb[b], isems[b])
            pltpu.async_copy(rows_h.at[pl.ds(off, C)], rowb[b], isems[b])
            pltpu.async_copy(vals_h.at[pl.ds(off, C)], valb[b], isems[b])

        def wait_idx(b):
            for dst in (colb[b], rowb[b], valb[b]):
                pltpu.make_async_copy(cols_h.at[pl.ds(0, C)], dst,
                                      isems[b]).wait()

        gbufs = (g0, g1)
        ssems = (ss0, ss1)

        def scale_chunk(valv, g):
            nsl = f // LANES

            def grp_body(grp, carry):
                e0 = grp * 2
                e1 = e0 + 1
                vw0 = plsc.load_gather(valv,
                                       [jnp.broadcast_to(e0, (LANES,))])
                vw1 = plsc.load_gather(valv,
                                       [jnp.broadcast_to(e1, (LANES,))])
                t0 = [g[e0, pl.ds(j * LANES, LANES)] for j in range(nsl)]
                t1 = [g[e1, pl.ds(j * LANES, LANES)] for j in range(nsl)]
                for j in range(nsl):
                    g[e0, pl.ds(j * LANES, LANES)] = t0[j] * vw0
                for j in range(nsl):
                    g[e1, pl.ds(j * LANES, LANES)] = t1[j] * vw1
                return carry

            lax.fori_loop(0, C // 2, grp_body, 0)

        fetch_idx(0, 0)

        def half_body(p, b, i):
            @pl.when(p >= 1)
            def _():
                pltpu.make_async_copy(gbufs[b], acc.at[rowb[b]],
                                      ssems[b]).wait()
            wait_idx(b)
            pltpu.async_copy(table.at[colb[b]], gbufs[b], sem).wait()
            if b == 0:
                fetch_idx(i + 1, 1)
            else:
                @pl.when(i + 1 < nchunk)
                def _():
                    fetch_idx(i + 1, 0)
            scale_chunk(valb[b], gbufs[b])
            pltpu.async_copy(gbufs[b], acc.at[rowb[b]], ssems[b], add=True)

        def pair_body(p, carry):
            half_body(p, 0, 2 * p)
            half_body(p, 1, 2 * p + 1)
            return carry

        lax.fori_loop(0, nchunk // 2, pair_body, 0)
        if nchunk % 2:
            pltpu.make_async_copy(g0, acc.at[rowb[0]], ss0).wait()
            wait_idx(0)
            pltpu.async_copy(table.at[colb[0]], g0, sem).wait()
            scale_chunk(valb[0], g0)
            pltpu.sync_copy(g0, acc.at[rowb[0]], add=True)
            pltpu.make_async_copy(g1, acc.at[rowb[1]], ss1).wait()
        else:
            pltpu.make_async_copy(g0, acc.at[rowb[0]], ss0).wait()
            pltpu.make_async_copy(g1, acc.at[rowb[1]], ss1).wait()
        plsc.subcore_barrier()

        pltpu.sync_copy(acc.at[pl.ds(s * rpw, rpw)],
                        out.at[pl.ds(c * n_pad + s * rpw, rpw)])

    return spmm, n_pad


def _tc1(x, p0, p1, w0, w1, inv_n):
    n, f = x.shape
    blk = 1000

    def body(xr, p0r, p1r, w0r, w1r, z1r, y01r):
        z1 = (p0r[...] + p1r[...]) * inv_n
        z1r[...] = z1
        y01r[...] = (jnp.dot(xr[...], w0r[...],
                             preferred_element_type=jnp.float32)
                     + jnp.dot(z1, w1r[...],
                               preferred_element_type=jnp.float32))

    row_spec = pl.BlockSpec((blk, f), lambda i: (i, 0))
    w_spec = pl.BlockSpec((f, f), lambda i: (0, 0))
    return pl.pallas_call(
        body,
        grid=(n // blk,),
        in_specs=[row_spec, row_spec, row_spec, w_spec, w_spec],
        out_specs=[row_spec, row_spec],
        out_shape=[jax.ShapeDtypeStruct((n, f), jnp.float32),
                   jax.ShapeDtypeStruct((n, f), jnp.float32)],
    )(x, p0, p1, w0, w1)


def _tc2(y01, q0, q1, w2, inv_n):
    n, f = y01.shape
    blk = 1000

    def body(y01r, q0r, q1r, w2r, yr):
        z2 = (q0r[...] + q1r[...]) * inv_n
        yr[...] = y01r[...] + jnp.dot(z2, w2r[...],
                                      preferred_element_type=jnp.float32)

    row_spec = pl.BlockSpec((blk, f), lambda i: (i, 0))
    w_spec = pl.BlockSpec((f, f), lambda i: (0, 0))
    return pl.pallas_call(
        body,
        grid=(n // blk,),
        in_specs=[row_spec, row_spec, row_spec, w_spec],
        out_specs=row_spec,
        out_shape=jax.ShapeDtypeStruct((n, f), jnp.float32),
    )(y01, q0, q1, w2)


def kernel(x, edge_index, edge_weight, weights):
    n, f = x.shape
    e = edge_weight.shape[0]
    rows = edge_index[0]
    cols = edge_index[1]
    inv_n = float(1.0 / n)

    unit = NW * CHUNK * 2
    e_pad = -(-e // unit) * unit
    pad = e_pad - e
    cols = jnp.concatenate([cols, jnp.zeros((pad,), cols.dtype)])
    rows = jnp.concatenate([rows, jnp.zeros((pad,), rows.dtype)])
    edge_weight = jnp.concatenate(
        [edge_weight, jnp.zeros((pad,), edge_weight.dtype)])

    spmm, n_pad = _make_spmm(n, e_pad, f)
    p = spmm(x, cols, rows, edge_weight)
    z1, y01 = _tc1(x, p[:n], p[n_pad:n_pad + n], weights[0], weights[1],
                   inv_n)
    q = spmm(z1, cols, rows, edge_weight)
    return _tc2(y01, q[:n], q[n_pad:n_pad + n], weights[2], inv_n)


# speedup vs baseline: 2.6599x; 1.9926x over previous
import functools

import jax
import jax.numpy as jnp
from jax import lax
from jax.experimental import pallas as pl
from jax.experimental.pallas import tpu as pltpu
from jax.experimental.pallas import tpu_sc as plsc

NC = 2
NS = 16
NW = NC * NS
LANES = 16
CHUNK = 128


def _make_spmm(n, e, f):
    epw = e // NW
    C = CHUNK
    nchunk = epw // C
    n_pad = -(-n // 128) * 128
    rpw = n_pad // NS
    mesh = plsc.VectorSubcoreMesh(core_axis_name="c", subcore_axis_name="s")

    @functools.partial(
        pl.kernel,
        out_type=jax.ShapeDtypeStruct((2 * n_pad, f), jnp.float32),
        mesh=mesh,
        compiler_params=pltpu.CompilerParams(needs_layout_passes=False),
        scratch_types=[
            pltpu.VMEM((C,), jnp.int32),
            pltpu.VMEM((C,), jnp.int32),
            pltpu.VMEM((C,), jnp.float32),
            pltpu.VMEM((C,), jnp.int32),
            pltpu.VMEM((C,), jnp.int32),
            pltpu.VMEM((C,), jnp.float32),
            pltpu.VMEM((C, f), jnp.float32),
            pltpu.VMEM((C, f), jnp.float32),
            pltpu.VMEM_SHARED((n_pad, f), jnp.float32),
            pltpu.SemaphoreType.DMA,
            pltpu.SemaphoreType.DMA,
            pltpu.SemaphoreType.DMA,
            pltpu.SemaphoreType.DMA,
            pltpu.SemaphoreType.DMA,
        ],
    )
    def spmm(table, cols_h, rows_h, vals_h, out, cv0, rv0, vv0,
             cv1, rv1, vv1, g0, g1, acc, sem, is0, is1, ss0, ss1):
        c = lax.axis_index("c")
        s = lax.axis_index("s")
        wid = s * NC + c

        z16 = jnp.zeros((LANES,), jnp.float32)

        def zrow(r, carry):
            for j in range(f // LANES):
                g0[r, pl.ds(j * LANES, LANES)] = z16
            return carry

        lax.fori_loop(0, C, zrow, 0)
        nzc = rpw // C
        rem = rpw - nzc * C
        for q in range(nzc):
            pltpu.sync_copy(g0, acc.at[pl.ds(s * rpw + q * C, C)])
        if rem:
            pltpu.sync_copy(g0.at[pl.ds(0, rem)],
                            acc.at[pl.ds(s * rpw + nzc * C, rem)])
        plsc.subcore_barrier()

        base = wid * epw
        colb = (cv0, cv1)
        rowb = (rv0, rv1)
        valb = (vv0, vv1)
        isems = (is0, is1)

        def fetch_idx(i, b):
            off = base + i * C
            pltpu.async_copy(cols_h.at[pl.ds(off, C)], colb[b], isems[b])
            pltpu.async_copy(rows_h.at[pl.ds(off, C)], rowb[b], isems[b])
            pltpu.async_copy(vals_h.at[pl.ds(off, C)], valb[b], isems[b])

        def wait_idx(b):
            for dst in (colb[b], rowb[b], valb[b]):
                pltpu.make_async_copy(cols_h.at[pl.ds(0, C)], dst,
                                      isems[b]).wait()

        gbufs = (g0, g1)
        ssems = (ss0, ss1)

        def scale_chunk(valv, g):
            nsl = f // LANES

            def grp_body(grp, carry):
                e0 = grp * 2
                e1 = e0 + 1
                vw0 = plsc.load_gather(valv,
                                       [jnp.broadcast_to(e0, (LANES,))])
                vw1 = plsc.load_gather(valv,
                                       [jnp.broadcast_to(e1, (LANES,))])
                t0 = [g[e0, pl.ds(j * LANES, LANES)] for j in range(nsl)]
                t1 = [g[e1, pl.ds(j * LANES, LANES)] for j in range(nsl)]
                for j in range(nsl):
                    g[e0, pl.ds(j * LANES, LANES)] = t0[j] * vw0
                for j in range(nsl):
                    g[e1, pl.ds(j * LANES, LANES)] = t1[j] * vw1
                return carry

            lax.fori_loop(0, C // 2, grp_body, 0)

        fetch_idx(0, 0)

        def half_body(p, b, i):
            @pl.when(p >= 1)
            def _():
                pltpu.make_async_copy(gbufs[b], acc.at[rowb[b]],
                                      ssems[b]).wait()
            wait_idx(b)
            pltpu.async_copy(table.at[colb[b]], gbufs[b], sem).wait()
            if b == 0:
                fetch_idx(i + 1, 1)
            else:
                @pl.when(i + 1 < nchunk)
                def _():
                    fetch_idx(i + 1, 0)
            scale_chunk(valb[b], gbufs[b])
            pltpu.async_copy(gbufs[b], acc.at[rowb[b]], ssems[b], add=True)

        def pair_body(p, carry):
            half_body(p, 0, 2 * p)
            half_body(p, 1, 2 * p + 1)
            return carry

        lax.fori_loop(0, nchunk // 2, pair_body, 0)
        if nchunk % 2:
            pltpu.make_async_copy(g0, acc.at[rowb[0]], ss0).wait()
            wait_idx(0)
            pltpu.async_copy(table.at[colb[0]], g0, sem).wait()
            scale_chunk(valb[0], g0)
            pltpu.sync_copy(g0, acc.at[rowb[0]], add=True)
            pltpu.make_async_copy(g1, acc.at[rowb[1]], ss1).wait()
        else:
            pltpu.make_async_copy(g0, acc.at[rowb[0]], ss0).wait()
            pltpu.make_async_copy(g1, acc.at[rowb[1]], ss1).wait()
        plsc.subcore_barrier()

        pltpu.sync_copy(acc.at[pl.ds(s * rpw, rpw)],
                        out.at[pl.ds(c * n_pad + s * rpw, rpw)])

    return spmm, n_pad


def _tc1(x, p0, p1, w0, w1, inv_n):
    n, f = x.shape
    blk = 1000

    def body(xr, p0r, p1r, w0r, w1r, z1r, y01r):
        z1 = (p0r[...] + p1r[...]) * inv_n
        z1r[...] = z1
        y01r[...] = (jnp.dot(xr[...], w0r[...],
                             preferred_element_type=jnp.float32)
                     + jnp.dot(z1, w1r[...],
                               preferred_element_type=jnp.float32))

    row_spec = pl.BlockSpec((blk, f), lambda i: (i, 0))
    w_spec = pl.BlockSpec((f, f), lambda i: (0, 0))
    return pl.pallas_call(
        body,
        grid=(n // blk,),
        in_specs=[row_spec, row_spec, row_spec, w_spec, w_spec],
        out_specs=[row_spec, row_spec],
        out_shape=[jax.ShapeDtypeStruct((n, f), jnp.float32),
                   jax.ShapeDtypeStruct((n, f), jnp.float32)],
    )(x, p0, p1, w0, w1)


def _tc2(y01, q0, q1, w2, inv_n):
    n, f = y01.shape
    blk = 1000

    def body(y01r, q0r, q1r, w2r, yr):
        z2 = (q0r[...] + q1r[...]) * inv_n
        yr[...] = y01r[...] + jnp.dot(z2, w2r[...],
                                      preferred_element_type=jnp.float32)

    row_spec = pl.BlockSpec((blk, f), lambda i: (i, 0))
    w_spec = pl.BlockSpec((f, f), lambda i: (0, 0))
    return pl.pallas_call(
        body,
        grid=(n // blk,),
        in_specs=[row_spec, row_spec, row_spec, w_spec],
        out_specs=row_spec,
        out_shape=jax.ShapeDtypeStruct((n, f), jnp.float32),
    )(y01, q0, q1, w2)


def kernel(x, edge_index, edge_weight, weights):
    n, f = x.shape
    e = edge_weight.shape[0]
    rows = edge_index[0]
    cols = edge_index[1]
    inv_n = float(1.0 / n)

    unit = NW * CHUNK * 2
    e_pad = -(-e // unit) * unit
    pad = e_pad - e
    spread = (jnp.arange(pad, dtype=cols.dtype) * 97) % n
    cols = jnp.concatenate([cols, spread])
    rows = jnp.concatenate([rows, spread])
    edge_weight = jnp.concatenate(
        [edge_weight, jnp.zeros((pad,), edge_weight.dtype)])

    spmm, n_pad = _make_spmm(n, e_pad, f)
    p = spmm(x, cols, rows, edge_weight)
    z1, y01 = _tc1(x, p[:n], p[n_pad:n_pad + n], weights[0], weights[1],
                   inv_n)
    q = spmm(z1, cols, rows, edge_weight)
    return _tc2(y01, q[:n], q[n_pad:n_pad + n], weights[2], inv_n)
